# Initial kernel scaffold; baseline (speedup 1.0000x reference)
import jax
import jax.numpy as jnp
from jax.experimental import pallas as pl


def kernel(x, fc, bias):
    raise NotImplementedError("write your pallas kernel here")


# speedup vs baseline: 1.2497x; 1.2497x over previous
import functools

import jax
import jax.numpy as jnp
from jax import lax
from jax.experimental import pallas as pl
from jax.experimental.pallas import tpu as pltpu
from jax.experimental.pallas import tpu_sc as plsc

_FIELDS = 26
_BATCH = 16384
_FIELD_SIZE = 40000
_NC = 2
_NS = 16
_NW = _NC * _NS
_BPW = _BATCH // _NW
_EPW = _BPW * _FIELDS
_IDX_MINOR = 128
_IDX_ROWS = _EPW // _IDX_MINOR
_L = 16
_JCH = _BPW // _L


def _body(x_hbm, fc_hbm, bias_hbm, out_hbm, xv, idxv, gv, outv, biasv, sem):
    wid = lax.axis_index("s") * _NC + lax.axis_index("c")
    base = wid * _BPW
    ebase = wid * _EPW

    pltpu.sync_copy(x_hbm.at[pl.ds(ebase, _EPW)], xv)
    pltpu.sync_copy(bias_hbm, biasv)

    lane = lax.iota(jnp.int32, 16)

    def build(r, carry):
        f = r // 4
        blk = r % 4
        foff = f * _FIELD_SIZE
        for c in range(8):
            b16 = blk * _IDX_MINOR + c * _L + lane
            src = b16 * _FIELDS + f
            v = plsc.load_gather(xv, [src])
            idxv[r, pl.ds(c * _L, _L)] = v + foff
        return carry

    lax.fori_loop(0, _IDX_ROWS, build, 0, unroll=False)

    def fire(r, carry):
        pltpu.async_copy(fc_hbm.at[idxv.at[r]], gv.at[r], sem)
        return carry

    lax.fori_loop(0, _IDX_ROWS, fire, 0, unroll=False)

    def drain(r, carry):
        pltpu.make_async_copy(fc_hbm.at[idxv.at[r]], gv.at[r], sem).wait()
        return carry

    lax.fori_loop(0, _IDX_ROWS, drain, 0, unroll=False)

    bias_splat = biasv[pl.ds(0, _L)]

    def accum(j, carry):
        jdiv = j // 8
        jcol = (j % 8) * _L
        acc = bias_splat
        for f in range(_FIELDS):
            acc = acc + gv[f * 4 + jdiv, pl.ds(jcol, _L)]
        outv[pl.ds(j * _L, _L)] = acc
        return carry

    lax.fori_loop(0, _JCH, accum, 0, unroll=False)

    pltpu.sync_copy(outv, out_hbm.at[pl.ds(base, _BPW)])


@jax.jit
def kernel(x, fc, bias):
    xf = x.reshape(-1)
    fcf = fc.reshape(-1)
    bias16 = jnp.broadcast_to(bias.reshape(()), (_L,)).astype(jnp.float32)
    mesh = plsc.VectorSubcoreMesh(core_axis_name="c", subcore_axis_name="s")
    run = functools.partial(
        pl.kernel,
        mesh=mesh,
        compiler_params=pltpu.CompilerParams(needs_layout_passes=False),
        out_type=jax.ShapeDtypeStruct((_BATCH,), jnp.float32),
        scratch_types=[
            pltpu.VMEM((_EPW,), jnp.int32),
            pltpu.VMEM((_IDX_ROWS, _IDX_MINOR), jnp.int32),
            pltpu.VMEM((_IDX_ROWS, _IDX_MINOR), jnp.float32),
            pltpu.VMEM((_BPW,), jnp.float32),
            pltpu.VMEM((_L,), jnp.float32),
            pltpu.SemaphoreType.DMA,
        ],
    )(_body)
    out = run(xf, fcf, bias16)
    return out.reshape(_BATCH, 1)
